# baseline (speedup 1.0000x reference)
import functools

import jax
import jax.numpy as jnp
import numpy as np
from jax import lax
from jax.experimental import pallas as pl
from jax.experimental.pallas import tpu as pltpu
from jax.experimental.pallas import tpu_sc as plsc

_COV_VALS = [
    0.2, 0.31, 0.28, 1.28, 0.96, 0.84, 0.76, 0.71, 0.66, 0.57,
    0.58, 1.66, 1.41, 1.21, 1.11, 1.07, 1.05, 1.02, 1.06, 2.03,
    1.76, 1.70, 1.60, 1.53, 1.39, 1.39, 1.32, 1.26, 1.24, 1.32,
    1.22, 1.22, 1.20, 1.19, 1.20, 1.20, 1.16, 2.20, 1.95, 1.90,
    1.75, 1.64, 1.54, 1.47, 1.46, 1.42, 1.39, 1.45, 1.44, 1.42,
    1.39, 1.39, 1.38, 1.39, 1.40, 2.44, 2.15, 2.07, 2.04, 2.03,
    2.01, 1.99, 1.98, 1.98, 1.96, 1.94, 1.92, 1.92, 1.89, 1.90,
    1.87, 1.87, 1.75, 1.70, 1.62, 1.51, 1.44, 1.41, 1.36, 1.36,
    1.32, 1.45, 1.46, 1.48, 1.40, 1.50, 1.50, 2.60, 2.21, 2.15,
    2.06, 2.00, 1.96, 1.90, 1.87, 1.80, 1.69, 0.2, 0.2, 0.2,
    0.2, 0.2, 0.2, 0.2, 0.2, 0.2, 0.2, 0.2, 0.2, 0.2,
    0.2, 0.2, 0.2, 0.2, 0.2, 0.2, 0.2, 0.2, 0.2,
]
_COV128 = np.zeros((128,), dtype=np.float32)
_COV128[: len(_COV_VALS)] = _COV_VALS

_N_NODES = 100000
_N_EDGES = 6400000
_N_ELEMS = 10
_CHUNK = 2000
_NCHUNK = 800
_N_NODE_CHUNKS = _N_NODES // _NCHUNK


def _make_fused_kernel():
    info = plsc.get_sparse_core_info()
    nc, ns = info.num_cores, info.num_subcores
    nw = nc * ns
    per_w = _N_EDGES // nw
    assert _N_EDGES % nw == 0 and per_w % _CHUNK == 0
    n_chunks = per_w // _CHUNK
    assert n_chunks % 2 == 0
    n_vec = _CHUNK // 16
    node_rounds = -(-_N_NODE_CHUNKS // ns)
    mesh = plsc.VectorSubcoreMesh(core_axis_name="c", subcore_axis_name="s")

    @functools.partial(
        pl.kernel,
        mesh=mesh,
        compiler_params=pltpu.CompilerParams(needs_layout_passes=False),
        out_type=(
            jax.ShapeDtypeStruct((_N_EDGES,), jnp.float32),
            jax.ShapeDtypeStruct((nc * _N_NODES,), jnp.float32),
        ),
        scratch_types=[
            pltpu.VMEM((_N_NODES,), jnp.float32),
            pltpu.VMEM((16,), jnp.int32),
            pltpu.VMEM((128,), jnp.float32),
            pltpu.VMEM((16,), jnp.float32),
            pltpu.VMEM((_NCHUNK * _N_ELEMS,), jnp.float32),
            pltpu.VMEM((_NCHUNK,), jnp.float32),
            pltpu.VMEM((_CHUNK,), jnp.int32),
            pltpu.VMEM((_CHUNK,), jnp.int32),
            pltpu.VMEM((_CHUNK,), jnp.float32),
            pltpu.VMEM((_CHUNK,), jnp.float32),
            pltpu.VMEM((_CHUNK,), jnp.int32),
            pltpu.VMEM((_CHUNK,), jnp.int32),
            pltpu.VMEM((_CHUNK,), jnp.float32),
            pltpu.VMEM((_CHUNK,), jnp.float32),
            pltpu.SemaphoreType.DMA,
            pltpu.SemaphoreType.DMA,
            pltpu.SemaphoreType.DMA,
            pltpu.SemaphoreType.DMA,
        ],
    )
    def fused_kernel(attrs_hbm, an_hbm, cov_hbm, src_hbm, tgt_hbm, x_hbm,
                     out_hbm, tabout_hbm,
                     tab_v, an_v, cov_v, covf_v, attrs_v, val_v,
                     src0, tgt0, x0, y0, src1, tgt1, x1, y1,
                     isem0, isem1, osem0, osem1):
        c = lax.axis_index("c")
        s = lax.axis_index("s")
        wid = s * nc + c
        base0 = wid * per_w

        bufs = ((src0, tgt0, x0, y0, isem0, osem0),
                (src1, tgt1, x1, y1, isem1, osem1))

        def in_copies(g, b):
            base = base0 + g * _CHUNK
            src_v, tgt_v, x_v, _, isem, _ = bufs[b]
            return (
                pltpu.make_async_copy(src_hbm.at[pl.ds(base, _CHUNK)], src_v, isem),
                pltpu.make_async_copy(tgt_hbm.at[pl.ds(base, _CHUNK)], tgt_v, isem),
                pltpu.make_async_copy(x_hbm.at[pl.ds(base, _CHUNK)], x_v, isem),
            )

        def out_copy(g, b):
            base = base0 + g * _CHUNK
            y_v, osem = bufs[b][3], bufs[b][5]
            return pltpu.make_async_copy(y_v, out_hbm.at[pl.ds(base, _CHUNK)], osem)

        def issue_in(g, b):
            for cp in in_copies(g, b):
                cp.start()

        issue_in(0, 0)
        issue_in(1, 1)

        pltpu.sync_copy(an_hbm, an_v)
        pltpu.sync_copy(cov_hbm, cov_v)
        covf = plsc.load_gather(cov_v, [an_v[...]])
        covf_v[...] = covf * 0.25

        def node_round(r, carry):
            ch = r * ns + s

            @pl.when(ch < _N_NODE_CHUNKS)
            def _():
                nbase = ch * _NCHUNK
                pltpu.sync_copy(
                    attrs_hbm.at[pl.ds(nbase * _N_ELEMS, _NCHUNK * _N_ELEMS)],
                    attrs_v)

                @plsc.parallel_loop(0, _NCHUNK // 16, unroll=4)
                def node_vec(j):
                    lane10 = lax.iota(jnp.int32, 16) * _N_ELEMS
                    base = j * (16 * _N_ELEMS)
                    m = plsc.load_gather(attrs_v, [lane10 + base])
                    amax = jnp.zeros((16,), jnp.int32)
                    for k in range(1, _N_ELEMS):
                        ak = plsc.load_gather(attrs_v, [lane10 + (base + k)])
                        isgt = ak > m
                        amax = jnp.where(isgt, k, amax)
                        m = jnp.maximum(ak, m)
                    val_v[pl.ds(j * 16, 16)] = plsc.load_gather(covf_v, [amax])

                pltpu.sync_copy(
                    val_v, tabout_hbm.at[pl.ds(c * _N_NODES + nbase, _NCHUNK)])
            return carry

        lax.fori_loop(0, node_rounds, node_round, 0)
        plsc.subcore_barrier()
        pltpu.sync_copy(tabout_hbm.at[pl.ds(c * _N_NODES, _N_NODES)], tab_v)

        def compute(b):
            src_v, tgt_v, x_v, y_v = bufs[b][0], bufs[b][1], bufs[b][2], bufs[b][3]

            @plsc.parallel_loop(0, n_vec, unroll=8)
            def vec_body(j):
                sl = pl.ds(j * 16, 16)
                gs = plsc.load_gather(tab_v, [src_v[sl]])
                gt = plsc.load_gather(tab_v, [tgt_v[sl]])
                xv = x_v[sl]
                r = xv / (gs + gt)
                p = r * -2.0 - (r * r * r) * 0.4
                e = jnp.exp(p)
                y_v[sl] = xv + e / (1.0 + e)

        def chunk_pair(g2, carry):
            for b in (0, 1):
                g = g2 * 2 + b
                for cp in in_copies(g, b):
                    cp.wait()

                @pl.when(g2 > 0)
                def _():
                    out_copy(g - 2, b).wait()

                compute(b)
                out_copy(g, b).start()

                @pl.when(g + 2 < n_chunks)
                def _():
                    issue_in(g + 2, b)
            return carry

        lax.fori_loop(0, n_chunks // 2, chunk_pair, 0)
        out_copy(n_chunks - 2, 0).wait()
        out_copy(n_chunks - 1, 1).wait()

    return fused_kernel


def kernel(x, node_attrs, edge_index, atomic_numbers):
    an16 = jnp.zeros((16,), jnp.int32).at[:_N_ELEMS].set(atomic_numbers)
    attrs_flat = node_attrs.reshape(_N_NODES * _N_ELEMS)
    src = edge_index[0]
    tgt = edge_index[1]
    xf = x.reshape(_N_EDGES)
    y, _ = _make_fused_kernel()(
        attrs_flat, an16, jnp.asarray(_COV128), src, tgt, xf)
    return y.reshape(_N_EDGES, 1)


# speedup vs baseline: 1.6331x; 1.6331x over previous
import functools

import jax
import jax.numpy as jnp
import numpy as np
from jax import lax
from jax.experimental import pallas as pl
from jax.experimental.pallas import tpu as pltpu
from jax.experimental.pallas import tpu_sc as plsc

_COV_VALS = [
    0.2, 0.31, 0.28, 1.28, 0.96, 0.84, 0.76, 0.71, 0.66, 0.57,
    0.58, 1.66, 1.41, 1.21, 1.11, 1.07, 1.05, 1.02, 1.06, 2.03,
    1.76, 1.70, 1.60, 1.53, 1.39, 1.39, 1.32, 1.26, 1.24, 1.32,
    1.22, 1.22, 1.20, 1.19, 1.20, 1.20, 1.16, 2.20, 1.95, 1.90,
    1.75, 1.64, 1.54, 1.47, 1.46, 1.42, 1.39, 1.45, 1.44, 1.42,
    1.39, 1.39, 1.38, 1.39, 1.40, 2.44, 2.15, 2.07, 2.04, 2.03,
    2.01, 1.99, 1.98, 1.98, 1.96, 1.94, 1.92, 1.92, 1.89, 1.90,
    1.87, 1.87, 1.75, 1.70, 1.62, 1.51, 1.44, 1.41, 1.36, 1.36,
    1.32, 1.45, 1.46, 1.48, 1.40, 1.50, 1.50, 2.60, 2.21, 2.15,
    2.06, 2.00, 1.96, 1.90, 1.87, 1.80, 1.69, 0.2, 0.2, 0.2,
    0.2, 0.2, 0.2, 0.2, 0.2, 0.2, 0.2, 0.2, 0.2, 0.2,
    0.2, 0.2, 0.2, 0.2, 0.2, 0.2, 0.2, 0.2, 0.2,
]
_COV = np.zeros((8, 128), dtype=np.float32)
_COV[0, : len(_COV_VALS)] = _COV_VALS

_N_NODES = 100000
_N_EDGES = 6400000
_N_ELEMS = 10
_COLS = 4000
_CHUNK = 2000


def _node_tab_body(an_ref, cov_ref, attrs_ref, out_ref):
    attrs = attrs_ref[0]
    m = jnp.max(attrs, axis=0, keepdims=True)
    k10 = lax.broadcasted_iota(jnp.int32, attrs.shape, 0)
    amax = jnp.min(jnp.where(attrs == m, k10, _N_ELEMS), axis=0, keepdims=True)
    an = an_ref[...]
    z128 = lax.broadcasted_iota(jnp.int32, an.shape, 1)
    cov = cov_ref[0:1, :]
    covf = jnp.sum(jnp.where(an == z128, cov, 0.0), axis=1, keepdims=True)
    covf10 = covf[0:_N_ELEMS]
    val = jnp.sum(jnp.where(amax == k10, covf10, 0.0), axis=0, keepdims=True)
    out_ref[0] = val * 0.25


def _node_tab(an_bc, attrs_t):
    out = pl.pallas_call(
        _node_tab_body,
        grid=(1,),
        in_specs=[
            pl.BlockSpec((16, 128), lambda i: (0, 0)),
            pl.BlockSpec((8, 128), lambda i: (0, 0)),
            pl.BlockSpec((1, _N_ELEMS, _N_NODES), lambda i: (0, 0, 0)),
        ],
        out_specs=pl.BlockSpec((1, 1, _N_NODES), lambda i: (0, 0, 0)),
        out_shape=jax.ShapeDtypeStruct((1, 1, _N_NODES), jnp.float32),
    )(an_bc, jnp.asarray(_COV), attrs_t)
    return out


def _make_edge_kernel():
    info = plsc.get_sparse_core_info()
    nc, ns = info.num_cores, info.num_subcores
    nw = nc * ns
    per_w = _N_EDGES // nw
    assert _N_EDGES % nw == 0 and per_w % _CHUNK == 0
    n_chunks = per_w // _CHUNK
    assert n_chunks % 2 == 0
    n_vec = _CHUNK // 16
    mesh = plsc.VectorSubcoreMesh(core_axis_name="c", subcore_axis_name="s")

    @functools.partial(
        pl.kernel,
        mesh=mesh,
        compiler_params=pltpu.CompilerParams(needs_layout_passes=False),
        out_type=jax.ShapeDtypeStruct((_N_EDGES,), jnp.float32),
        scratch_types=[
            pltpu.VMEM((_N_NODES,), jnp.float32),
            pltpu.VMEM((_CHUNK,), jnp.int32),
            pltpu.VMEM((_CHUNK,), jnp.int32),
            pltpu.VMEM((_CHUNK,), jnp.float32),
            pltpu.VMEM((_CHUNK,), jnp.float32),
            pltpu.VMEM((_CHUNK,), jnp.int32),
            pltpu.VMEM((_CHUNK,), jnp.int32),
            pltpu.VMEM((_CHUNK,), jnp.float32),
            pltpu.VMEM((_CHUNK,), jnp.float32),
            pltpu.SemaphoreType.DMA,
            pltpu.SemaphoreType.DMA,
            pltpu.SemaphoreType.DMA,
            pltpu.SemaphoreType.DMA,
            pltpu.SemaphoreType.DMA,
        ],
    )
    def edge_kernel(tab_hbm, ei_hbm, x_hbm, out_hbm,
                    tab_v,
                    src0, tgt0, x0, y0, src1, tgt1, x1, y1,
                    isem0, isem1, osem0, osem1, tsem):
        c = lax.axis_index("c")
        s = lax.axis_index("s")
        wid = s * nc + c
        base0 = wid * per_w
        tab_cp = pltpu.make_async_copy(tab_hbm, tab_v, tsem)
        tab_cp.start()

        bufs = ((src0, tgt0, x0, y0, isem0, osem0),
                (src1, tgt1, x1, y1, isem1, osem1))

        def in_copies(g, b):
            base = base0 + g * _CHUNK
            src_v, tgt_v, x_v, _, isem, _ = bufs[b]
            return (
                pltpu.make_async_copy(ei_hbm.at[pl.ds(base, _CHUNK)], src_v, isem),
                pltpu.make_async_copy(
                    ei_hbm.at[pl.ds(_N_EDGES + base, _CHUNK)], tgt_v, isem),
                pltpu.make_async_copy(x_hbm.at[pl.ds(base, _CHUNK)], x_v, isem),
            )

        def out_copy(g, b):
            base = base0 + g * _CHUNK
            y_v, osem = bufs[b][3], bufs[b][5]
            return pltpu.make_async_copy(y_v, out_hbm.at[pl.ds(base, _CHUNK)], osem)

        def issue_in(g, b):
            for cp in in_copies(g, b):
                cp.start()

        def compute(b):
            src_v, tgt_v, x_v, y_v = bufs[b][0], bufs[b][1], bufs[b][2], bufs[b][3]

            @plsc.parallel_loop(0, n_vec, unroll=8)
            def vec_body(j):
                sl = pl.ds(j * 16, 16)
                gs = plsc.load_gather(tab_v, [src_v[sl]])
                gt = plsc.load_gather(tab_v, [tgt_v[sl]])
                xv = x_v[sl]
                r = xv / (gs + gt)
                p = r * -2.0 - (r * r * r) * 0.4
                e = jnp.exp(p)
                y_v[sl] = xv + e / (1.0 + e)

        issue_in(0, 0)
        issue_in(1, 1)
        tab_cp.wait()

        def chunk_pair(g2, carry):
            for b in (0, 1):
                g = g2 * 2 + b
                for cp in in_copies(g, b):
                    cp.wait()

                @pl.when(g2 > 0)
                def _():
                    out_copy(g - 2, b).wait()

                compute(b)
                out_copy(g, b).start()

                @pl.when(g + 2 < n_chunks)
                def _():
                    issue_in(g + 2, b)
            return carry

        lax.fori_loop(0, n_chunks // 2, chunk_pair, 0)
        out_copy(n_chunks - 2, 0).wait()
        out_copy(n_chunks - 1, 1).wait()

    return edge_kernel


def kernel(x, node_attrs, edge_index, atomic_numbers):
    an_bc = jnp.full((16, 128), -1, jnp.int32).at[:_N_ELEMS, :].set(
        jnp.broadcast_to(atomic_numbers[:, None], (_N_ELEMS, 128)))
    attrs_t = node_attrs.T.reshape(1, _N_ELEMS, _N_NODES)
    tab = _node_tab(an_bc, attrs_t).reshape(_N_NODES)
    ei_flat = edge_index.reshape(2 * _N_EDGES)
    xf = x.reshape(_N_EDGES)
    y = _make_edge_kernel()(tab, ei_flat, xf)
    return y.reshape(_N_EDGES, 1)


# speedup vs baseline: 2.2378x; 1.3702x over previous
import functools

import jax
import jax.numpy as jnp
import numpy as np
from jax import lax
from jax.experimental import pallas as pl
from jax.experimental.pallas import tpu as pltpu
from jax.experimental.pallas import tpu_sc as plsc

_COV_VALS = [
    0.2, 0.31, 0.28, 1.28, 0.96, 0.84, 0.76, 0.71, 0.66, 0.57,
    0.58, 1.66, 1.41, 1.21, 1.11, 1.07, 1.05, 1.02, 1.06, 2.03,
    1.76, 1.70, 1.60, 1.53, 1.39, 1.39, 1.32, 1.26, 1.24, 1.32,
    1.22, 1.22, 1.20, 1.19, 1.20, 1.20, 1.16, 2.20, 1.95, 1.90,
    1.75, 1.64, 1.54, 1.47, 1.46, 1.42, 1.39, 1.45, 1.44, 1.42,
    1.39, 1.39, 1.38, 1.39, 1.40, 2.44, 2.15, 2.07, 2.04, 2.03,
    2.01, 1.99, 1.98, 1.98, 1.96, 1.94, 1.92, 1.92, 1.89, 1.90,
    1.87, 1.87, 1.75, 1.70, 1.62, 1.51, 1.44, 1.41, 1.36, 1.36,
    1.32, 1.45, 1.46, 1.48, 1.40, 1.50, 1.50, 2.60, 2.21, 2.15,
    2.06, 2.00, 1.96, 1.90, 1.87, 1.80, 1.69, 0.2, 0.2, 0.2,
    0.2, 0.2, 0.2, 0.2, 0.2, 0.2, 0.2, 0.2, 0.2, 0.2,
    0.2, 0.2, 0.2, 0.2, 0.2, 0.2, 0.2, 0.2, 0.2,
]
_COV = np.zeros((8, 128), dtype=np.float32)
_COV[0, : len(_COV_VALS)] = _COV_VALS

_N_NODES = 100000
_N_EDGES = 6400000
_N_ELEMS = 10
_COLS = 4000
_CHUNK = 2048


def _node_tab_body(an_ref, cov_ref, attrs_ref, out_ref):
    attrs = attrs_ref[0]
    m = jnp.max(attrs, axis=0, keepdims=True)
    k10 = lax.broadcasted_iota(jnp.int32, attrs.shape, 0)
    amax = jnp.min(jnp.where(attrs == m, k10, _N_ELEMS), axis=0, keepdims=True)
    an = an_ref[...]
    z128 = lax.broadcasted_iota(jnp.int32, an.shape, 1)
    cov = cov_ref[0:1, :]
    covf = jnp.sum(jnp.where(an == z128, cov, 0.0), axis=1, keepdims=True)
    covf10 = covf[0:_N_ELEMS]
    val = jnp.sum(jnp.where(amax == k10, covf10, 0.0), axis=0, keepdims=True)
    out_ref[0] = val * 0.25


def _node_tab(an_bc, attrs_t):
    out = pl.pallas_call(
        _node_tab_body,
        grid=(1,),
        in_specs=[
            pl.BlockSpec((16, 128), lambda i: (0, 0)),
            pl.BlockSpec((8, 128), lambda i: (0, 0)),
            pl.BlockSpec((1, _N_ELEMS, _N_NODES), lambda i: (0, 0, 0)),
        ],
        out_specs=pl.BlockSpec((1, 1, _N_NODES), lambda i: (0, 0, 0)),
        out_shape=jax.ShapeDtypeStruct((1, 1, _N_NODES), jnp.float32),
    )(an_bc, jnp.asarray(_COV), attrs_t)
    return out


def _make_edge_kernel():
    info = plsc.get_sparse_core_info()
    nc, ns = info.num_cores, info.num_subcores
    nw = nc * ns
    total_chunks = _N_EDGES // _CHUNK
    assert _N_EDGES % _CHUNK == 0 and _CHUNK % 128 == 0
    n_lo = total_chunks // nw
    n_extra = total_chunks % nw
    n_hi = n_lo + (1 if n_extra else 0)
    pairs = -(-n_hi // 2)
    n_vec = _CHUNK // 16
    mesh = plsc.VectorSubcoreMesh(core_axis_name="c", subcore_axis_name="s")

    @functools.partial(
        pl.kernel,
        mesh=mesh,
        compiler_params=pltpu.CompilerParams(needs_layout_passes=False),
        out_type=jax.ShapeDtypeStruct((_N_EDGES,), jnp.float32),
        scratch_types=[
            pltpu.VMEM((_N_NODES,), jnp.float32),
            pltpu.VMEM((2, _CHUNK), jnp.int32),
            pltpu.VMEM((_CHUNK,), jnp.float32),
            pltpu.VMEM((_CHUNK,), jnp.float32),
            pltpu.VMEM((2, _CHUNK), jnp.int32),
            pltpu.VMEM((_CHUNK,), jnp.float32),
            pltpu.VMEM((_CHUNK,), jnp.float32),
            pltpu.SemaphoreType.DMA,
            pltpu.SemaphoreType.DMA,
            pltpu.SemaphoreType.DMA,
            pltpu.SemaphoreType.DMA,
            pltpu.SemaphoreType.DMA,
        ],
    )
    def edge_kernel(tab_hbm, ei_hbm, x_hbm, out_hbm,
                    tab_v,
                    ei0, x0, y0, ei1, x1, y1,
                    isem0, isem1, osem0, osem1, tsem):
        c = lax.axis_index("c")
        s = lax.axis_index("s")
        wid = s * nc + c
        n_my = n_lo + jnp.where(wid < n_extra, 1, 0)
        tab_cp = pltpu.make_async_copy(tab_hbm, tab_v, tsem)
        tab_cp.start()

        bufs = ((ei0, x0, y0, isem0, osem0),
                (ei1, x1, y1, isem1, osem1))

        def in_copies(g, b):
            base = (wid + nw * g) * _CHUNK
            ei_v, x_v, _, isem, _ = bufs[b]
            return (
                pltpu.make_async_copy(ei_hbm.at[:, pl.ds(base, _CHUNK)], ei_v, isem),
                pltpu.make_async_copy(x_hbm.at[pl.ds(base, _CHUNK)], x_v, isem),
            )

        def out_copy(g, b):
            base = (wid + nw * g) * _CHUNK
            y_v, osem = bufs[b][2], bufs[b][4]
            return pltpu.make_async_copy(y_v, out_hbm.at[pl.ds(base, _CHUNK)], osem)

        def issue_in(g, b):
            for cp in in_copies(g, b):
                cp.start()

        def compute(b):
            ei_v, x_v, y_v = bufs[b][0], bufs[b][1], bufs[b][2]

            @plsc.parallel_loop(0, n_vec, unroll=8)
            def vec_body(j):
                sl = pl.ds(j * 16, 16)
                gs = plsc.load_gather(tab_v, [ei_v[0, sl]])
                gt = plsc.load_gather(tab_v, [ei_v[1, sl]])
                xv = x_v[sl]
                r = xv / (gs + gt)
                p = r * -2.0 - (r * r * r) * 0.4
                e = jnp.exp(p)
                y_v[sl] = xv + e / (1.0 + e)

        issue_in(0, 0)
        issue_in(1, 1)
        tab_cp.wait()

        def chunk_pair(g2, carry):
            for b in (0, 1):
                g = g2 * 2 + b

                @pl.when(g < n_my)
                def _():
                    for cp in in_copies(g, b):
                        cp.wait()

                    @pl.when(g2 > 0)
                    def _():
                        out_copy(g - 2, b).wait()

                    compute(b)
                    out_copy(g, b).start()

                    @pl.when(g + 2 < n_my)
                    def _():
                        issue_in(g + 2, b)
            return carry

        lax.fori_loop(0, pairs, chunk_pair, 0)
        even = n_my % 2 == 0
        out_copy(jnp.where(even, n_my - 2, n_my - 1), 0).wait()
        out_copy(jnp.where(even, n_my - 1, n_my - 2), 1).wait()

    return edge_kernel


def kernel(x, node_attrs, edge_index, atomic_numbers):
    an_bc = jnp.full((16, 128), -1, jnp.int32).at[:_N_ELEMS, :].set(
        jnp.broadcast_to(atomic_numbers[:, None], (_N_ELEMS, 128)))
    attrs_t = node_attrs.T.reshape(1, _N_ELEMS, _N_NODES)
    tab = _node_tab(an_bc, attrs_t).reshape(_N_NODES)
    xf = x.reshape(_N_EDGES)
    y = _make_edge_kernel()(tab, edge_index, xf)
    return y.reshape(_N_EDGES, 1)


# speedup vs baseline: 2.5011x; 1.1177x over previous
import functools

import jax
import jax.numpy as jnp
import numpy as np
from jax import lax
from jax.experimental import pallas as pl
from jax.experimental.pallas import tpu as pltpu
from jax.experimental.pallas import tpu_sc as plsc

_COV_VALS = [
    0.2, 0.31, 0.28, 1.28, 0.96, 0.84, 0.76, 0.71, 0.66, 0.57,
    0.58, 1.66, 1.41, 1.21, 1.11, 1.07, 1.05, 1.02, 1.06, 2.03,
    1.76, 1.70, 1.60, 1.53, 1.39, 1.39, 1.32, 1.26, 1.24, 1.32,
    1.22, 1.22, 1.20, 1.19, 1.20, 1.20, 1.16, 2.20, 1.95, 1.90,
    1.75, 1.64, 1.54, 1.47, 1.46, 1.42, 1.39, 1.45, 1.44, 1.42,
    1.39, 1.39, 1.38, 1.39, 1.40, 2.44, 2.15, 2.07, 2.04, 2.03,
    2.01, 1.99, 1.98, 1.98, 1.96, 1.94, 1.92, 1.92, 1.89, 1.90,
    1.87, 1.87, 1.75, 1.70, 1.62, 1.51, 1.44, 1.41, 1.36, 1.36,
    1.32, 1.45, 1.46, 1.48, 1.40, 1.50, 1.50, 2.60, 2.21, 2.15,
    2.06, 2.00, 1.96, 1.90, 1.87, 1.80, 1.69, 0.2, 0.2, 0.2,
    0.2, 0.2, 0.2, 0.2, 0.2, 0.2, 0.2, 0.2, 0.2, 0.2,
    0.2, 0.2, 0.2, 0.2, 0.2, 0.2, 0.2, 0.2, 0.2,
]
_COV = np.zeros((8, 128), dtype=np.float32)
_COV[0, : len(_COV_VALS)] = _COV_VALS

_N_NODES = 100000
_N_EDGES = 6400000
_N_ELEMS = 10
_COLS = 4000
_CHUNK = 3200


def _node_tab_body(an_ref, cov_ref, attrs_ref, out_ref):
    attrs = attrs_ref[0]
    m = jnp.max(attrs, axis=0, keepdims=True)
    k10 = lax.broadcasted_iota(jnp.int32, attrs.shape, 0)
    amax = jnp.min(jnp.where(attrs == m, k10, _N_ELEMS), axis=0, keepdims=True)
    an = an_ref[...]
    z128 = lax.broadcasted_iota(jnp.int32, an.shape, 1)
    cov = cov_ref[0:1, :]
    covf = jnp.sum(jnp.where(an == z128, cov, 0.0), axis=1, keepdims=True)
    covf10 = covf[0:_N_ELEMS]
    val = jnp.sum(jnp.where(amax == k10, covf10, 0.0), axis=0, keepdims=True)
    out_ref[0] = val * 0.25


def _node_tab(an_bc, attrs_t):
    out = pl.pallas_call(
        _node_tab_body,
        grid=(1,),
        in_specs=[
            pl.BlockSpec((16, 128), lambda i: (0, 0)),
            pl.BlockSpec((8, 128), lambda i: (0, 0)),
            pl.BlockSpec((1, _N_ELEMS, _N_NODES), lambda i: (0, 0, 0)),
        ],
        out_specs=pl.BlockSpec((1, 1, _N_NODES), lambda i: (0, 0, 0)),
        out_shape=jax.ShapeDtypeStruct((1, 1, _N_NODES), jnp.float32),
    )(an_bc, jnp.asarray(_COV), attrs_t)
    return out


def _make_edge_kernel():
    info = plsc.get_sparse_core_info()
    nc, ns = info.num_cores, info.num_subcores
    nw = nc * ns
    total_chunks = _N_EDGES // _CHUNK
    assert _N_EDGES % _CHUNK == 0 and _CHUNK % 128 == 0
    n_lo = total_chunks // nw
    n_extra = total_chunks % nw
    n_hi = n_lo + (1 if n_extra else 0)
    pairs = -(-n_hi // 2)
    n_vec = _CHUNK // 16
    mesh = plsc.VectorSubcoreMesh(core_axis_name="c", subcore_axis_name="s")

    @functools.partial(
        pl.kernel,
        mesh=mesh,
        compiler_params=pltpu.CompilerParams(needs_layout_passes=False),
        out_type=jax.ShapeDtypeStruct((_N_EDGES,), jnp.float32),
        scratch_types=[
            pltpu.VMEM((_N_NODES,), jnp.float32),
            pltpu.VMEM((2, _CHUNK), jnp.int32),
            pltpu.VMEM((_CHUNK,), jnp.float32),
            pltpu.VMEM((_CHUNK,), jnp.float32),
            pltpu.VMEM((2, _CHUNK), jnp.int32),
            pltpu.VMEM((_CHUNK,), jnp.float32),
            pltpu.VMEM((_CHUNK,), jnp.float32),
            pltpu.SemaphoreType.DMA,
            pltpu.SemaphoreType.DMA,
            pltpu.SemaphoreType.DMA,
            pltpu.SemaphoreType.DMA,
            pltpu.SemaphoreType.DMA,
        ],
    )
    def edge_kernel(tab_hbm, ei_hbm, x_hbm, out_hbm,
                    tab_v,
                    ei0, x0, y0, ei1, x1, y1,
                    isem0, isem1, osem0, osem1, tsem):
        c = lax.axis_index("c")
        s = lax.axis_index("s")
        wid = s * nc + c
        n_my = n_lo + jnp.where(wid < n_extra, 1, 0)
        tab_cp = pltpu.make_async_copy(tab_hbm, tab_v, tsem)
        tab_cp.start()

        bufs = ((ei0, x0, y0, isem0, osem0),
                (ei1, x1, y1, isem1, osem1))

        def in_copies(g, b):
            base = (wid + nw * g) * _CHUNK
            ei_v, x_v, _, isem, _ = bufs[b]
            return (
                pltpu.make_async_copy(ei_hbm.at[:, pl.ds(base, _CHUNK)], ei_v, isem),
                pltpu.make_async_copy(x_hbm.at[pl.ds(base, _CHUNK)], x_v, isem),
            )

        def out_copy(g, b):
            base = (wid + nw * g) * _CHUNK
            y_v, osem = bufs[b][2], bufs[b][4]
            return pltpu.make_async_copy(y_v, out_hbm.at[pl.ds(base, _CHUNK)], osem)

        def issue_in(g, b):
            for cp in in_copies(g, b):
                cp.start()

        def compute(b):
            ei_v, x_v, y_v = bufs[b][0], bufs[b][1], bufs[b][2]

            @plsc.parallel_loop(0, n_vec, unroll=8)
            def vec_body(j):
                sl = pl.ds(j * 16, 16)
                gs = plsc.load_gather(tab_v, [ei_v[0, sl]])
                gt = plsc.load_gather(tab_v, [ei_v[1, sl]])
                xv = x_v[sl]
                r = xv / (gs + gt)
                p = r * -2.0 - (r * r * r) * 0.4
                e = jnp.exp(p)
                y_v[sl] = xv + e / (1.0 + e)

        issue_in(0, 0)
        issue_in(1, 1)
        tab_cp.wait()

        def chunk_pair(g2, carry):
            for b in (0, 1):
                g = g2 * 2 + b

                @pl.when(g < n_my)
                def _():
                    for cp in in_copies(g, b):
                        cp.wait()

                    @pl.when(g2 > 0)
                    def _():
                        out_copy(g - 2, b).wait()

                    compute(b)
                    out_copy(g, b).start()

                    @pl.when(g + 2 < n_my)
                    def _():
                        issue_in(g + 2, b)
            return carry

        lax.fori_loop(0, pairs, chunk_pair, 0)
        even = n_my % 2 == 0
        out_copy(jnp.where(even, n_my - 2, n_my - 1), 0).wait()
        out_copy(jnp.where(even, n_my - 1, n_my - 2), 1).wait()

    return edge_kernel


def kernel(x, node_attrs, edge_index, atomic_numbers):
    an_bc = jnp.full((16, 128), -1, jnp.int32).at[:_N_ELEMS, :].set(
        jnp.broadcast_to(atomic_numbers[:, None], (_N_ELEMS, 128)))
    attrs_t = node_attrs.T.reshape(1, _N_ELEMS, _N_NODES)
    tab = _node_tab(an_bc, attrs_t).reshape(_N_NODES)
    xf = x.reshape(_N_EDGES)
    y = _make_edge_kernel()(tab, edge_index, xf)
    return y.reshape(_N_EDGES, 1)
